# baseline (speedup 1.0000x reference)
import jax
import jax.numpy as jnp
from jax.experimental import pallas as pl
from jax.experimental.pallas import tpu as pltpu
from jax.experimental.pallas import tpu_sc as plsc

_B, _L, _C, _H = 64, 576, 384, 24
_OVERLAP, _NUM_VQS = 4, 6
_CB_DIM, _CB_SIZE = 256, 8192
_W_SP = _L // _H
_T = _W_SP // _OVERLAP
_FIX_DIM = _H * _C
_GROUP_DIM = (_OVERLAP * _FIX_DIM) // _NUM_VQS
_ROWS = _B * _T
_EPS = 1e-12
_K_TILE = 1024
_NUM_KT = _CB_SIZE // _K_TILE
_NUM_IDX = _NUM_VQS * _ROWS
_GW = 128


def _proj_body(x_ref, w_ref, o_ref):
    x = x_ref[...]
    w = w_ref[0]
    ze = jnp.dot(x, w, preferred_element_type=jnp.float32)
    n = jnp.sqrt(jnp.sum(ze * ze, axis=1, keepdims=True))
    o_ref[0] = ze / (n + _EPS)


def _argmax_body(zen_ref, cb_ref, idx_ref, bestv_ref, besti_ref):
    g = pl.program_id(0)
    k = pl.program_id(1)
    zen = zen_ref[0]
    tile = cb_ref[0]
    rn = jnp.sqrt(jnp.sum(tile * tile, axis=1, keepdims=True))
    tile_n = tile / (rn + _EPS)
    sim = jax.lax.dot_general(
        zen, tile_n, (((1,), (1,)), ((), ())),
        preferred_element_type=jnp.float32)
    m = jnp.max(sim, axis=1, keepdims=True)
    lane = jax.lax.broadcasted_iota(jnp.int32, (_ROWS, _K_TILE), 1)
    big = jnp.int32(2**30)
    loc = jnp.min(jnp.where(sim == m, lane, big), axis=1, keepdims=True)
    gidx = loc + (k * _K_TILE + g * _CB_SIZE)

    @pl.when(k == 0)
    def _():
        bestv_ref[...] = m
        besti_ref[...] = gidx

    @pl.when(k > 0)
    def _():
        upd = m > bestv_ref[...]
        bestv_ref[...] = jnp.where(upd, m, bestv_ref[...])
        besti_ref[...] = jnp.where(upd, gidx, besti_ref[...])

    @pl.when(k == _NUM_KT - 1)
    def _():
        idx_ref[0] = besti_ref[...]


def _out_body(zq_ref, zen_ref, w_ref, o_ref, loss_ref):
    g = pl.program_id(0)
    zq = zq_ref[0]
    n = jnp.sqrt(jnp.sum(zq * zq, axis=1, keepdims=True))
    zqn = zq / (n + _EPS)
    w = w_ref[0]
    o_ref[...] = jnp.dot(zqn, w, preferred_element_type=jnp.float32)
    diff = zen_ref[0] - zqn
    rs = jnp.sum(diff * diff, axis=1)
    lane = jax.lax.broadcasted_iota(jnp.int32, (_B, _ROWS), 1)
    sub = jax.lax.broadcasted_iota(jnp.int32, (_B, _ROWS), 0)
    mask = (lane // _T) == sub
    contrib = jnp.sum(jnp.where(mask, rs[None, :], 0.0), axis=1)
    contrib = contrib * (1.0 / (_T * _CB_DIM * _NUM_VQS))

    @pl.when(g == 0)
    def _():
        loss_ref[0] = contrib

    @pl.when(g > 0)
    def _():
        loss_ref[0] = loss_ref[0] + contrib


def _sc_gather(cb_flat, idx_flat):
    mesh = plsc.VectorSubcoreMesh(core_axis_name="c", subcore_axis_name="s")

    @pl.kernel(
        out_type=jax.ShapeDtypeStruct((_NUM_IDX, _CB_DIM), jnp.float32),
        mesh=mesh)
    def kern(x_hbm, i_hbm, o_hbm):
        def body(i_vmem, o_vmem):
            pltpu.sync_copy(x_hbm.at[i_vmem.at[0]], o_vmem)

        pltpu.emit_pipeline(
            body,
            grid=(_NUM_IDX // _GW,),
            in_specs=[pl.BlockSpec((1, _GW), index_map=lambda i: (0, i))],
            out_specs=[pl.BlockSpec((_GW, _CB_DIM), index_map=lambda i: (i, 0))],
            core_axis_name="s",
            dimension_semantics=(pltpu.PARALLEL,),
        )(i_hbm, o_hbm)

    return kern(cb_flat, idx_flat)


def kernel(z, Win, codebooks, Wout):
    z4 = z.reshape(_B, _H, _W_SP, _C)
    zt = jnp.transpose(z4, (0, 2, 3, 1)).reshape(_B, _T, _OVERLAP * _FIX_DIM)
    zp = zt.reshape(_ROWS, _NUM_VQS * _GROUP_DIM)

    zen = pl.pallas_call(
        _proj_body,
        grid=(_NUM_VQS,),
        in_specs=[
            pl.BlockSpec((_ROWS, _GROUP_DIM), lambda g: (0, g)),
            pl.BlockSpec((1, _GROUP_DIM, _CB_DIM), lambda g: (g, 0, 0)),
        ],
        out_specs=pl.BlockSpec((1, _ROWS, _CB_DIM), lambda g: (g, 0, 0)),
        out_shape=jax.ShapeDtypeStruct((_NUM_VQS, _ROWS, _CB_DIM), jnp.float32),
    )(zp, Win)

    idx = pl.pallas_call(
        _argmax_body,
        grid=(_NUM_VQS, _NUM_KT),
        in_specs=[
            pl.BlockSpec((1, _ROWS, _CB_DIM), lambda g, k: (g, 0, 0)),
            pl.BlockSpec((1, _K_TILE, _CB_DIM), lambda g, k: (g, k, 0)),
        ],
        out_specs=pl.BlockSpec((1, _ROWS, 1), lambda g, k: (g, 0, 0)),
        out_shape=jax.ShapeDtypeStruct((_NUM_VQS, _ROWS, 1), jnp.int32),
        scratch_shapes=[
            pltpu.VMEM((_ROWS, 1), jnp.float32),
            pltpu.VMEM((_ROWS, 1), jnp.int32),
        ],
    )(zen, codebooks)

    cb_flat = codebooks.reshape(_NUM_VQS * _CB_SIZE, _CB_DIM)
    idx_flat = idx.reshape(1, _NUM_IDX)
    zq_raw = _sc_gather(cb_flat, idx_flat).reshape(_NUM_VQS, _ROWS, _CB_DIM)

    zq_cols, loss = pl.pallas_call(
        _out_body,
        grid=(_NUM_VQS,),
        in_specs=[
            pl.BlockSpec((1, _ROWS, _CB_DIM), lambda g: (g, 0, 0)),
            pl.BlockSpec((1, _ROWS, _CB_DIM), lambda g: (g, 0, 0)),
            pl.BlockSpec((1, _CB_DIM, _GROUP_DIM), lambda g: (g, 0, 0)),
        ],
        out_specs=[
            pl.BlockSpec((_ROWS, _GROUP_DIM), lambda g: (0, g)),
            pl.BlockSpec((1, _B), lambda g: (0, 0)),
        ],
        out_shape=[
            jax.ShapeDtypeStruct((_ROWS, _NUM_VQS * _GROUP_DIM), jnp.float32),
            jax.ShapeDtypeStruct((1, _B), jnp.float32),
        ],
    )(zq_raw, zen, Wout)

    zq3 = zq_cols.reshape(_B, _T, _OVERLAP * _FIX_DIM)
    zq2 = zq3.reshape(_B, _W_SP, _FIX_DIM).reshape(_B, _W_SP, _C, _H)
    out = jnp.transpose(zq2, (0, 3, 1, 2)).reshape(_B, _L, _C)
    lossv = loss.reshape(_B)
    return out, lossv, lossv
